# Initial kernel scaffold; baseline (speedup 1.0000x reference)
import jax
import jax.numpy as jnp
from jax.experimental import pallas as pl


def kernel(x, W1, b1, W2, b2, Wf1, bf1, Wf2, bf2, routes):
    raise NotImplementedError("write your pallas kernel here")


# speedup vs baseline: 42.6309x; 42.6309x over previous
import functools
import math

import jax
import jax.numpy as jnp
from jax.experimental import pallas as pl
from jax.experimental.pallas import tpu as pltpu


def _routing_kernel(cap, routesT_ref, mw_ref):
    ne, t = mw_ref.shape
    ne -= 1
    r = routesT_ref[...]
    e = (r * ne) // t
    e0 = e[0:1, :]
    e1 = e[1:2, :]
    dup = e0 == e1
    iota_e = jax.lax.broadcasted_iota(jnp.int32, (ne, t), 0)
    m0 = (iota_e == e0).astype(jnp.float32)
    m1e = (iota_e == e1).astype(jnp.float32)
    m1 = jnp.where(dup, 0.0, m1e)
    ctot = m0 + m1
    c = ctot
    sh = 1
    while sh < t:
        shifted = jnp.concatenate(
            [jnp.zeros((ne, sh), jnp.float32), c[:, : t - sh]], axis=1)
        c = c + shifted
        sh *= 2
    excl = c - ctot
    pos0 = jnp.sum(excl * m0, axis=0, keepdims=True)
    pos1 = jnp.sum((excl + m0) * m1e, axis=0, keepdims=True)
    capf = jnp.float32(cap)
    acc0 = (pos0 < capf).astype(jnp.float32)
    acc1 = jnp.where(dup, 0.0, (pos1 < capf).astype(jnp.float32))
    used = acc0 + acc1
    invused = 1.0 / jnp.maximum(used, 1.0)
    macc = m0 * acc0 + m1 * acc1
    fb = (used == 0.0).astype(jnp.float32)
    mw_ref[...] = jnp.concatenate([macc * invused, fb], axis=0)


def _ffn_kernel(ne, tblk, mw_ref, x_ref, w1_ref, w2_ref, wf1_ref, wf2_ref,
                b1_ref, b2_ref, out_ref, h_ref):
    e = pl.program_id(0)
    f = pl.program_id(1)
    tb = pl.program_id(2)
    xb = x_ref[...]

    @pl.when(e < ne)
    def _():
        h_ref[...] = jnp.dot(xb, w1_ref[0],
                             preferred_element_type=jnp.float32)

    @pl.when(e == ne)
    def _():
        h_ref[...] = jnp.dot(xb, wf1_ref[...],
                             preferred_element_type=jnp.float32)

    b1b = b1_ref[0]
    hpre = h_ref[...] + b1b
    h = hpre * 0.5 * (1.0 + jax.lax.erf(hpre * 0.7071067811865476))

    def second(w_mat):
        return jnp.dot(h, w_mat, preferred_element_type=jnp.float32)

    scale = mw_ref[e, pl.ds(tb * tblk, tblk)]
    rows = pl.ds(tb * tblk, tblk)

    def emit(y):
        part = y * scale[:, None]

        @pl.when(f == 0)
        def _():
            part2 = part + scale[:, None] * b2_ref[0]

            @pl.when(e == 0)
            def _():
                out_ref[rows, :] = part2

            @pl.when(e != 0)
            def _():
                out_ref[rows, :] = out_ref[rows, :] + part2

        @pl.when(f != 0)
        def _():
            out_ref[rows, :] = out_ref[rows, :] + part

    @pl.when(e < ne)
    def _():
        emit(second(w2_ref[0]))

    @pl.when(e == ne)
    def _():
        emit(second(wf2_ref[...]))


def _moe_forward(x, w1, b1, w2, b2, wf1, bf1, wf2, bf2, routes,
                 tblk=None, fblk=None):
    t, d = x.shape
    ne, _, dff = w1.shape
    k = routes.shape[1]
    cap = int(1.25 * math.ceil(t * k / ne))
    tblk = tblk or min(512, t)
    fblk = fblk or min(1536, dff)
    ntb = t // tblk
    nfb = dff // fblk

    routesT = routes.astype(jnp.int32).T
    mw = pl.pallas_call(
        functools.partial(_routing_kernel, cap),
        out_shape=jax.ShapeDtypeStruct((ne + 1, t), jnp.float32),
    )(routesT)

    b1c = jnp.concatenate([b1, bf1[None]], axis=0).reshape(ne + 1, 1, dff)
    b2c = jnp.concatenate([b2, bf2[None]], axis=0).reshape(ne + 1, 1, d)

    grid = (ne + 1, nfb, ntb)
    out = pl.pallas_call(
        functools.partial(_ffn_kernel, ne, tblk),
        grid=grid,
        in_specs=[
            pl.BlockSpec((ne + 1, t), lambda e, f, tb: (0, 0)),
            pl.BlockSpec((tblk, d), lambda e, f, tb: (tb, 0)),
            pl.BlockSpec((1, d, fblk),
                         lambda e, f, tb: (jnp.minimum(e, ne - 1), 0,
                                           jnp.where(e < ne, f, 0))),
            pl.BlockSpec((1, fblk, d),
                         lambda e, f, tb: (jnp.minimum(e, ne - 1),
                                           jnp.where(e < ne, f, 0), 0)),
            pl.BlockSpec((d, fblk),
                         lambda e, f, tb: (0, jnp.where(e < ne, 0, f))),
            pl.BlockSpec((fblk, d),
                         lambda e, f, tb: (jnp.where(e < ne, 0, f), 0)),
            pl.BlockSpec((1, 1, fblk), lambda e, f, tb: (e, 0, f)),
            pl.BlockSpec((1, 1, d), lambda e, f, tb: (e, 0, 0)),
        ],
        out_specs=pl.BlockSpec((t, d), lambda e, f, tb: (0, 0)),
        out_shape=jax.ShapeDtypeStruct((t, d), jnp.float32),
        scratch_shapes=[pltpu.VMEM((tblk, fblk), jnp.float32)],
    )(mw, x, w1, w2, wf1, wf2, b1c, b2c)
    return out


def kernel(x, W1, b1, W2, b2, Wf1, bf1, Wf2, bf2, routes):
    return _moe_forward(x, W1, b1, W2, b2, Wf1, bf1, Wf2, bf2, routes)


# speedup vs baseline: 77.0365x; 1.8071x over previous
import functools
import math

import jax
import jax.numpy as jnp
from jax import lax
from jax.experimental import pallas as pl
from jax.experimental.pallas import tpu as pltpu
from jax.experimental.pallas import tpu_sc as plsc


def _routing_kernel(cap, capp, nwork, routesT_ref, slots_ref, wf_ref, loads_ref):
    ne = loads_ref.shape[0]
    t = routesT_ref.shape[1]
    r = routesT_ref[...]
    e = (r * ne) // t
    e0 = e[0:1, :]
    e1 = e[1:2, :]
    dup = e0 == e1
    iota_e = jax.lax.broadcasted_iota(jnp.int32, (ne, t), 0)
    m0 = (iota_e == e0).astype(jnp.float32)
    m1e = (iota_e == e1).astype(jnp.float32)
    m1 = jnp.where(dup, 0.0, m1e)
    ctot = m0 + m1
    c = ctot
    sh = 1
    while sh < t:
        c = c + jnp.concatenate(
            [jnp.zeros((ne, sh), jnp.float32), c[:, : t - sh]], axis=1)
        sh *= 2
    excl = c - ctot
    pos0 = jnp.sum(excl * m0, axis=0, keepdims=True)
    pos1 = jnp.sum((excl + m0) * m1e, axis=0, keepdims=True)
    capf = jnp.float32(cap)
    acc0 = pos0 < capf
    acc1 = jnp.logical_and(jnp.logical_not(dup), pos1 < capf)
    a0f = acc0.astype(jnp.float32)
    a1f = acc1.astype(jnp.float32)
    used = a0f + a1f
    invused = 1.0 / jnp.maximum(used, 1.0)
    fb = (used == 0.0).astype(jnp.float32)
    s0 = e0 * capp + pos0.astype(jnp.int32)
    s1 = e1 * capp + pos1.astype(jnp.int32)
    tok = jax.lax.broadcasted_iota(jnp.int32, (1, t), 1)
    trash = ne * capp + tok // (t // nwork)
    s0_scat = jnp.where(acc0, s0, trash)
    s1_scat = jnp.where(acc1, s1, trash)
    zero = jnp.zeros_like(s0)
    s0_gath = jnp.where(acc0, s0, jnp.where(acc1, s1, zero))
    s1_gath = jnp.where(acc1, s1, jnp.where(acc0, s0, zero))
    slots_ref[...] = jnp.concatenate([s0_scat, s1_scat, s0_gath, s1_gath], 0)
    wf_ref[...] = jnp.concatenate([a0f * invused, a1f * invused, fb], 0)
    loads_ref[...] = jnp.minimum(jnp.sum(ctot, axis=1, keepdims=True), capf)


def _sc_dispatch(nrows_pad, x, s0, s1):
    t, d = x.shape
    info = plsc.get_sparse_core_info()
    nc, ns = info.num_cores, info.num_subcores
    nw = nc * ns
    bpw = t // nw
    mesh = plsc.VectorSubcoreMesh(core_axis_name="c", subcore_axis_name="s")

    @functools.partial(
        pl.kernel, mesh=mesh,
        out_type=jax.ShapeDtypeStruct((nrows_pad, d), jnp.float32),
        scratch_types=[
            pltpu.VMEM((bpw, d), jnp.float32),
            pltpu.VMEM((bpw,), jnp.int32),
        ],
    )
    def k(x_hbm, s0_hbm, s1_hbm, xb_hbm, rows_v, idx_v):
        wid = lax.axis_index("s") * nc + lax.axis_index("c")
        base = wid * bpw
        pltpu.sync_copy(x_hbm.at[pl.ds(base, bpw)], rows_v)
        pltpu.sync_copy(s0_hbm.at[pl.ds(base, bpw)], idx_v)
        pltpu.sync_copy(rows_v, xb_hbm.at[idx_v])
        pltpu.sync_copy(s1_hbm.at[pl.ds(base, bpw)], idx_v)
        pltpu.sync_copy(rows_v, xb_hbm.at[idx_v])

    return k(x, s0, s1)


def _sc_gather2(yb, s0, s1):
    t = s0.shape[0]
    d = yb.shape[1]
    info = plsc.get_sparse_core_info()
    nc, ns = info.num_cores, info.num_subcores
    nw = nc * ns
    bpw = t // nw
    mesh = plsc.VectorSubcoreMesh(core_axis_name="c", subcore_axis_name="s")

    @functools.partial(
        pl.kernel, mesh=mesh,
        out_type=[jax.ShapeDtypeStruct((t, d), jnp.float32),
                  jax.ShapeDtypeStruct((t, d), jnp.float32)],
        scratch_types=[
            pltpu.VMEM((bpw, d), jnp.float32),
            pltpu.VMEM((bpw,), jnp.int32),
        ],
    )
    def k(yb_hbm, s0_hbm, s1_hbm, g0_hbm, g1_hbm, rows_v, idx_v):
        wid = lax.axis_index("s") * nc + lax.axis_index("c")
        base = wid * bpw
        pltpu.sync_copy(s0_hbm.at[pl.ds(base, bpw)], idx_v)
        pltpu.sync_copy(yb_hbm.at[idx_v], rows_v)
        pltpu.sync_copy(rows_v, g0_hbm.at[pl.ds(base, bpw)])
        pltpu.sync_copy(s1_hbm.at[pl.ds(base, bpw)], idx_v)
        pltpu.sync_copy(yb_hbm.at[idx_v], rows_v)
        pltpu.sync_copy(rows_v, g1_hbm.at[pl.ds(base, bpw)])

    return k(yb, s0, s1)


def _ffn_kernel(xb_ref, w1_ref, w2_ref, b1_ref, b2_ref, yb_ref,
                w1c_ref, w2c_ref):
    cb = pl.program_id(1)

    @pl.when(cb == 0)
    def _():
        w1c_ref[...] = w1_ref[0].astype(jnp.bfloat16)
        w2c_ref[...] = w2_ref[0].astype(jnp.bfloat16)

    xb = xb_ref[...].astype(jnp.bfloat16)
    hpre = jnp.dot(xb, w1c_ref[...],
                   preferred_element_type=jnp.float32) + b1_ref[0]
    h = (hpre * 0.5 * (1.0 + jax.lax.erf(hpre * 0.7071067811865476))
         ).astype(jnp.bfloat16)
    yb_ref[...] = jnp.dot(h, w2c_ref[...],
                          preferred_element_type=jnp.float32) + b2_ref[0]


def _expert_ffn(xb, w1, b1, w2, b2, capp, capb):
    ne, d, dff = w1.shape
    nrows = ne * capp
    ncb = capp // capb
    b1r = b1.reshape(ne, 1, dff)
    b2r = b2.reshape(ne, 1, d)
    return pl.pallas_call(
        _ffn_kernel,
        grid=(ne, ncb),
        in_specs=[
            pl.BlockSpec((capb, d), lambda e, cb: (e * (capp // capb) + cb, 0)),
            pl.BlockSpec((1, d, dff), lambda e, cb: (e, 0, 0)),
            pl.BlockSpec((1, dff, d), lambda e, cb: (e, 0, 0)),
            pl.BlockSpec((1, 1, dff), lambda e, cb: (e, 0, 0)),
            pl.BlockSpec((1, 1, d), lambda e, cb: (e, 0, 0)),
        ],
        out_specs=pl.BlockSpec((capb, d),
                               lambda e, cb: (e * (capp // capb) + cb, 0)),
        out_shape=jax.ShapeDtypeStruct((nrows, d), jnp.float32),
        scratch_shapes=[pltpu.VMEM((d, dff), jnp.bfloat16),
                        pltpu.VMEM((dff, d), jnp.bfloat16)],
    )(xb, w1, w2, b1r, b2r)


def _fbc_kernel(x_ref, wf1_ref, wf2_ref, bf1_ref, bf2_ref,
                g0_ref, g1_ref, wfb_ref, out_ref):
    xb = x_ref[...].astype(jnp.bfloat16)
    hpre = jnp.dot(xb, wf1_ref[...].astype(jnp.bfloat16),
                   preferred_element_type=jnp.float32) + bf1_ref[0]
    h = (hpre * 0.5 * (1.0 + jax.lax.erf(hpre * 0.7071067811865476))
         ).astype(jnp.bfloat16)
    yf = jnp.dot(h, wf2_ref[...].astype(jnp.bfloat16),
                 preferred_element_type=jnp.float32) + bf2_ref[0]
    w0 = wfb_ref[0, 0, 0, :][:, None]
    w1 = wfb_ref[0, 1, 0, :][:, None]
    fb = wfb_ref[0, 2, 0, :][:, None]
    g = g0_ref[...] * w0 + g1_ref[...] * w1
    out_ref[...] = jnp.where(fb > 0.0, yf, g)


def _fallback_combine(x, wf1, bf1, wf2, bf2, g0, g1, wfb, tblk):
    t, d = x.shape
    dff = wf1.shape[1]
    ntb = t // tblk
    wfbr = wfb.reshape(3, ntb, tblk).transpose(1, 0, 2).reshape(ntb, 3, 1, tblk)
    bf1r = bf1.reshape(1, 1, dff)
    bf2r = bf2.reshape(1, 1, d)
    return pl.pallas_call(
        _fbc_kernel,
        grid=(ntb,),
        in_specs=[
            pl.BlockSpec((tblk, d), lambda tb: (tb, 0)),
            pl.BlockSpec((d, dff), lambda tb: (0, 0)),
            pl.BlockSpec((dff, d), lambda tb: (0, 0)),
            pl.BlockSpec((1, 1, dff), lambda tb: (0, 0, 0)),
            pl.BlockSpec((1, 1, d), lambda tb: (0, 0, 0)),
            pl.BlockSpec((tblk, d), lambda tb: (tb, 0)),
            pl.BlockSpec((tblk, d), lambda tb: (tb, 0)),
            pl.BlockSpec((1, 3, 1, tblk), lambda tb: (tb, 0, 0, 0)),
        ],
        out_specs=pl.BlockSpec((tblk, d), lambda tb: (tb, 0)),
        out_shape=jax.ShapeDtypeStruct((t, d), jnp.float32),
    )(x, wf1, wf2, bf1r, bf2r, g0, g1, wfbr)


def _moe_forward(x, w1, b1, w2, b2, wf1, bf1, wf2, bf2, routes):
    t, d = x.shape
    ne, _, dff = w1.shape
    k = routes.shape[1]
    cap = int(1.25 * math.ceil(t * k / ne))
    capp = 768
    capb = 256
    nwork = 32
    nrows = ne * capp
    nrows_pad = nrows + 256

    routesT = routes.astype(jnp.int32).T
    slots, wfb, _loads = pl.pallas_call(
        functools.partial(_routing_kernel, cap, capp, nwork),
        out_shape=[jax.ShapeDtypeStruct((4, t), jnp.int32),
                   jax.ShapeDtypeStruct((3, t), jnp.float32),
                   jax.ShapeDtypeStruct((ne, 1), jnp.float32)],
    )(routesT)

    s0_scat = slots[0]
    s1_scat = slots[1]
    s0_gath = slots[2]
    s1_gath = slots[3]

    xb = _sc_dispatch(nrows_pad, x, s0_scat, s1_scat)
    yb = _expert_ffn(xb, w1, b1, w2, b2, capp, capb)
    g0, g1 = _sc_gather2(yb, s0_gath, s1_gath)
    out = _fallback_combine(x, wf1, bf1, wf2, bf2, g0, g1, wfb, tblk=512)
    return out


def kernel(x, W1, b1, W2, b2, Wf1, bf1, Wf2, bf2, routes):
    return _moe_forward(x, W1, b1, W2, b2, Wf1, bf1, Wf2, bf2, routes)


# speedup vs baseline: 110.3778x; 1.4328x over previous
import functools
import math

import jax
import jax.numpy as jnp
from jax import lax
from jax.experimental import pallas as pl
from jax.experimental.pallas import tpu as pltpu
from jax.experimental.pallas import tpu_sc as plsc


def _routing_kernel(cap, capp, nwork, routesT_ref, slots_ref, wf_ref, loads_ref):
    ne = loads_ref.shape[0]
    t = routesT_ref.shape[1]
    r = routesT_ref[...]
    e = (r * ne) // t
    e0 = e[0:1, :]
    e1 = e[1:2, :]
    dup = e0 == e1
    iota_e = jax.lax.broadcasted_iota(jnp.int32, (ne, t), 0)
    m0 = (iota_e == e0).astype(jnp.float32)
    m1e = (iota_e == e1).astype(jnp.float32)
    m1 = jnp.where(dup, 0.0, m1e)
    ctot = m0 + m1
    c = ctot
    sh = 1
    while sh < t:
        c = c + jnp.concatenate(
            [jnp.zeros((ne, sh), jnp.float32), c[:, : t - sh]], axis=1)
        sh *= 2
    excl = c - ctot
    pos0 = jnp.sum(excl * m0, axis=0, keepdims=True)
    pos1 = jnp.sum((excl + m0) * m1e, axis=0, keepdims=True)
    capf = jnp.float32(cap)
    acc0 = pos0 < capf
    acc1 = jnp.logical_and(jnp.logical_not(dup), pos1 < capf)
    a0f = acc0.astype(jnp.float32)
    a1f = acc1.astype(jnp.float32)
    used = a0f + a1f
    invused = 1.0 / jnp.maximum(used, 1.0)
    fb = (used == 0.0).astype(jnp.float32)
    s0 = e0 * capp + pos0.astype(jnp.int32)
    s1 = e1 * capp + pos1.astype(jnp.int32)
    tok = jax.lax.broadcasted_iota(jnp.int32, (1, t), 1)
    trash = ne * capp + tok // (t // nwork)
    s0_scat = jnp.where(acc0, s0, trash)
    s1_scat = jnp.where(acc1, s1, trash)
    zero = jnp.zeros_like(s0)
    s0_gath = jnp.where(acc0, s0, jnp.where(acc1, s1, zero))
    s1_gath = jnp.where(acc1, s1, jnp.where(acc0, s0, zero))
    slots_ref[...] = jnp.concatenate([s0_scat, s1_scat, s0_gath, s1_gath], 0)
    wf_ref[...] = jnp.concatenate([a0f * invused, a1f * invused, fb], 0)
    loads_ref[...] = jnp.minimum(jnp.sum(ctot, axis=1, keepdims=True), capf)


def _sc_dispatch(nrows_pad, x, slots):
    t, d = x.shape
    info = plsc.get_sparse_core_info()
    nc, ns = info.num_cores, info.num_subcores
    nw = nc * ns
    bpw = t // nw
    mesh = plsc.VectorSubcoreMesh(core_axis_name="c", subcore_axis_name="s")

    @functools.partial(
        pl.kernel, mesh=mesh,
        out_type=jax.ShapeDtypeStruct((nrows_pad, d), jnp.float32),
        scratch_types=[
            pltpu.VMEM((bpw, d), jnp.float32),
            pltpu.VMEM((bpw,), jnp.int32),
        ],
    )
    def k(x_hbm, slots_hbm, xb_hbm, rows_v, idx_v):
        wid = lax.axis_index("s") * nc + lax.axis_index("c")
        base = wid * bpw
        pltpu.sync_copy(x_hbm.at[pl.ds(base, bpw)], rows_v)
        pltpu.sync_copy(slots_hbm.at[0, pl.ds(base, bpw)], idx_v)
        pltpu.sync_copy(rows_v, xb_hbm.at[idx_v])
        pltpu.sync_copy(slots_hbm.at[1, pl.ds(base, bpw)], idx_v)
        pltpu.sync_copy(rows_v, xb_hbm.at[idx_v])

    return k(x, slots)


def _sc_gather2(yb, slots):
    t = slots.shape[1]
    d = yb.shape[1]
    info = plsc.get_sparse_core_info()
    nc, ns = info.num_cores, info.num_subcores
    nw = nc * ns
    bpw = t // nw
    mesh = plsc.VectorSubcoreMesh(core_axis_name="c", subcore_axis_name="s")

    @functools.partial(
        pl.kernel, mesh=mesh,
        out_type=[jax.ShapeDtypeStruct((t, d), jnp.float32),
                  jax.ShapeDtypeStruct((t, d), jnp.float32)],
        scratch_types=[
            pltpu.VMEM((bpw, d), jnp.float32),
            pltpu.VMEM((bpw,), jnp.int32),
        ],
    )
    def k(yb_hbm, slots_hbm, g0_hbm, g1_hbm, rows_v, idx_v):
        wid = lax.axis_index("s") * nc + lax.axis_index("c")
        base = wid * bpw
        pltpu.sync_copy(slots_hbm.at[2, pl.ds(base, bpw)], idx_v)
        pltpu.sync_copy(yb_hbm.at[idx_v], rows_v)
        pltpu.sync_copy(rows_v, g0_hbm.at[pl.ds(base, bpw)])
        pltpu.sync_copy(slots_hbm.at[3, pl.ds(base, bpw)], idx_v)
        pltpu.sync_copy(yb_hbm.at[idx_v], rows_v)
        pltpu.sync_copy(rows_v, g1_hbm.at[pl.ds(base, bpw)])

    return k(yb, slots)


def _ffn_kernel(capb, loads_ref, xb_ref, w1_ref, w2_ref, b1_ref, b2_ref,
                yb_ref, w1c_ref, w2c_ref):
    e = pl.program_id(0)
    cb = pl.program_id(1)
    needed = loads_ref[e] > cb * capb

    @pl.when(needed)
    def _():
        @pl.when(cb == 0)
        def _():
            w1c_ref[...] = w1_ref[0].astype(jnp.bfloat16)
            w2c_ref[...] = w2_ref[0].astype(jnp.bfloat16)

        xb = xb_ref[...].astype(jnp.bfloat16)
        hpre = jnp.dot(xb, w1c_ref[...],
                       preferred_element_type=jnp.float32) + b1_ref[0]
        h = (hpre * 0.5 * (1.0 + jax.lax.erf(hpre * 0.7071067811865476))
             ).astype(jnp.bfloat16)
        yb_ref[...] = jnp.dot(h, w2c_ref[...],
                              preferred_element_type=jnp.float32) + b2_ref[0]


def _expert_ffn(xb, w1, b1, w2, b2, loads, capp, capb):
    ne, d, dff = w1.shape
    nrows = ne * capp
    ncb = capp // capb
    b1r = b1.reshape(ne, 1, dff)
    b2r = b2.reshape(ne, 1, d)
    grid_spec = pltpu.PrefetchScalarGridSpec(
        num_scalar_prefetch=1,
        grid=(ne, ncb),
        in_specs=[
            pl.BlockSpec((capb, d),
                         lambda e, cb, L: (e * (capp // capb) + cb, 0)),
            pl.BlockSpec((1, d, dff),
                         lambda e, cb, L: (jnp.minimum(
                             e + (cb >= 1).astype(jnp.int32), ne - 1), 0, 0)),
            pl.BlockSpec((1, dff, d),
                         lambda e, cb, L: (jnp.minimum(
                             e + (cb >= 2).astype(jnp.int32), ne - 1), 0, 0)),
            pl.BlockSpec((1, 1, dff), lambda e, cb, L: (e, 0, 0)),
            pl.BlockSpec((1, 1, d), lambda e, cb, L: (e, 0, 0)),
        ],
        out_specs=pl.BlockSpec((capb, d),
                               lambda e, cb, L: (e * (capp // capb) + cb, 0)),
        scratch_shapes=[pltpu.VMEM((d, dff), jnp.bfloat16),
                        pltpu.VMEM((dff, d), jnp.bfloat16)],
    )
    return pl.pallas_call(
        functools.partial(_ffn_kernel, capb),
        grid_spec=grid_spec,
        out_shape=jax.ShapeDtypeStruct((nrows, d), jnp.float32),
    )(loads, xb, w1, w2, b1r, b2r)


def _fbc_kernel(nfb_ref, x_ref, wf1_ref, wf2_ref, bf1_ref, bf2_ref,
                g0_ref, g1_ref, wfb_ref, out_ref):
    tb = pl.program_id(0)
    w0 = wfb_ref[0, :][:, None]
    w1 = wfb_ref[1, :][:, None]
    fb = wfb_ref[2, :][:, None]
    g = g0_ref[...] * w0 + g1_ref[...] * w1

    @pl.when(nfb_ref[tb] > 0)
    def _():
        xb = x_ref[...].astype(jnp.bfloat16)
        hpre = jnp.dot(xb, wf1_ref[...].astype(jnp.bfloat16),
                       preferred_element_type=jnp.float32) + bf1_ref[0]
        h = (hpre * 0.5 * (1.0 + jax.lax.erf(hpre * 0.7071067811865476))
             ).astype(jnp.bfloat16)
        yf = jnp.dot(h, wf2_ref[...].astype(jnp.bfloat16),
                     preferred_element_type=jnp.float32) + bf2_ref[0]
        out_ref[...] = jnp.where(fb > 0.0, yf, g)

    @pl.when(nfb_ref[tb] == 0)
    def _():
        out_ref[...] = g


def _fallback_combine(x, wf1, bf1, wf2, bf2, g0, g1, wfb, nfb_blk, tblk):
    t, d = x.shape
    dff = wf1.shape[1]
    ntb = t // tblk
    bf1r = bf1.reshape(1, 1, dff)
    bf2r = bf2.reshape(1, 1, d)
    grid_spec = pltpu.PrefetchScalarGridSpec(
        num_scalar_prefetch=1,
        grid=(ntb,),
        in_specs=[
            pl.BlockSpec((tblk, d),
                         lambda tb, N: (jnp.where(N[tb] > 0, tb, 0), 0)),
            pl.BlockSpec((d, dff), lambda tb, N: (0, 0)),
            pl.BlockSpec((dff, d), lambda tb, N: (0, 0)),
            pl.BlockSpec((1, 1, dff), lambda tb, N: (0, 0, 0)),
            pl.BlockSpec((1, 1, d), lambda tb, N: (0, 0, 0)),
            pl.BlockSpec((tblk, d), lambda tb, N: (tb, 0)),
            pl.BlockSpec((tblk, d), lambda tb, N: (tb, 0)),
            pl.BlockSpec((3, tblk), lambda tb, N: (0, tb)),
        ],
        out_specs=pl.BlockSpec((tblk, d), lambda tb, N: (tb, 0)),
    )
    return pl.pallas_call(
        _fbc_kernel,
        grid_spec=grid_spec,
        out_shape=jax.ShapeDtypeStruct((t, d), jnp.float32),
    )(nfb_blk, x, wf1, wf2, bf1r, bf2r, g0, g1, wfb)


def _moe_forward(x, w1, b1, w2, b2, wf1, bf1, wf2, bf2, routes):
    t, d = x.shape
    ne, _, dff = w1.shape
    k = routes.shape[1]
    cap = int(1.25 * math.ceil(t * k / ne))
    capp = 768
    capb = 256
    nwork = 32
    nrows = ne * capp
    nrows_pad = nrows + 256

    routesT = routes.astype(jnp.int32).T
    slots, wfb, loads = pl.pallas_call(
        functools.partial(_routing_kernel, cap, capp, nwork),
        out_shape=[jax.ShapeDtypeStruct((4, t), jnp.int32),
                   jax.ShapeDtypeStruct((3, t), jnp.float32),
                   jax.ShapeDtypeStruct((ne, 1), jnp.float32)],
    )(routesT)

    loads_i = loads.astype(jnp.int32).reshape(ne)
    tblk = 512
    nfb_blk = jnp.sum(wfb[2].reshape(t // tblk, tblk),
                      axis=1).astype(jnp.int32)

    xb = _sc_dispatch(nrows_pad, x, slots)
    yb = _expert_ffn(xb, w1, b1, w2, b2, loads_i, capp, capb)
    g0, g1 = _sc_gather2(yb, slots)
    out = _fallback_combine(x, wf1, bf1, wf2, bf2, g0, g1, wfb, nfb_blk,
                            tblk=tblk)
    return out


def kernel(x, W1, b1, W2, b2, Wf1, bf1, Wf2, bf2, routes):
    return _moe_forward(x, W1, b1, W2, b2, Wf1, bf1, Wf2, bf2, routes)


# speedup vs baseline: 115.5213x; 1.0466x over previous
import functools
import math

import jax
import jax.numpy as jnp
from jax import lax
from jax.experimental import pallas as pl
from jax.experimental.pallas import tpu as pltpu
from jax.experimental.pallas import tpu_sc as plsc


def _routing_kernel(cap, capp, nwork, routesT_ref, slots_ref, wf_ref, loads_ref):
    ne = loads_ref.shape[0]
    t = routesT_ref.shape[1]
    r = routesT_ref[...]
    e = (r * ne) // t
    e0 = e[0:1, :]
    e1 = e[1:2, :]
    dup = e0 == e1
    iota_e = jax.lax.broadcasted_iota(jnp.int32, (ne, t), 0)
    m0 = (iota_e == e0).astype(jnp.float32)
    m1e = (iota_e == e1).astype(jnp.float32)
    m1 = jnp.where(dup, 0.0, m1e)
    ctot = m0 + m1
    c = ctot
    sh = 1
    while sh < t:
        c = c + jnp.concatenate(
            [jnp.zeros((ne, sh), jnp.float32), c[:, : t - sh]], axis=1)
        sh *= 2
    excl = c - ctot
    pos0 = jnp.sum(excl * m0, axis=0, keepdims=True)
    pos1 = jnp.sum((excl + m0) * m1e, axis=0, keepdims=True)
    capf = jnp.float32(cap)
    acc0 = pos0 < capf
    acc1 = jnp.logical_and(jnp.logical_not(dup), pos1 < capf)
    a0f = acc0.astype(jnp.float32)
    a1f = acc1.astype(jnp.float32)
    used = a0f + a1f
    invused = 1.0 / jnp.maximum(used, 1.0)
    fb = (used == 0.0).astype(jnp.float32)
    s0 = e0 * capp + pos0.astype(jnp.int32)
    s1 = e1 * capp + pos1.astype(jnp.int32)
    tok = jax.lax.broadcasted_iota(jnp.int32, (1, t), 1)
    trash = ne * capp + tok // (t // nwork)
    s0_scat = jnp.where(acc0, s0, trash)
    s1_scat = jnp.where(acc1, s1, trash)
    zero = jnp.zeros_like(s0)
    s0_gath = jnp.where(acc0, s0, jnp.where(acc1, s1, zero))
    s1_gath = jnp.where(acc1, s1, jnp.where(acc0, s0, zero))
    slots_ref[...] = jnp.concatenate([s0_scat, s1_scat, s0_gath, s1_gath], 0)
    wf_ref[...] = jnp.concatenate([a0f * invused, a1f * invused, fb], 0)
    loads_ref[...] = jnp.minimum(jnp.sum(ctot, axis=1, keepdims=True),
                                 capf).astype(jnp.int32)


def _sc_dispatch(nrows_pad, x, slots):
    t, d = x.shape
    info = plsc.get_sparse_core_info()
    nc, ns = info.num_cores, info.num_subcores
    nw = nc * ns
    bpw = t // nw
    mesh = plsc.VectorSubcoreMesh(core_axis_name="c", subcore_axis_name="s")

    @functools.partial(
        pl.kernel, mesh=mesh,
        out_type=jax.ShapeDtypeStruct((nrows_pad, d), jnp.float32),
        scratch_types=[
            pltpu.VMEM((bpw, d), jnp.float32),
            pltpu.VMEM((bpw,), jnp.int32),
        ],
    )
    def k(x_hbm, slots_hbm, xb_hbm, rows_v, idx_v):
        wid = lax.axis_index("s") * nc + lax.axis_index("c")
        base = wid * bpw
        pltpu.sync_copy(x_hbm.at[pl.ds(base, bpw)], rows_v)
        pltpu.sync_copy(slots_hbm.at[0, pl.ds(base, bpw)], idx_v)
        pltpu.sync_copy(rows_v, xb_hbm.at[idx_v])
        pltpu.sync_copy(slots_hbm.at[1, pl.ds(base, bpw)], idx_v)
        pltpu.sync_copy(rows_v, xb_hbm.at[idx_v])

    return k(x, slots)


def _sc_gather2(yb, slots):
    t = slots.shape[1]
    d = yb.shape[1]
    info = plsc.get_sparse_core_info()
    nc, ns = info.num_cores, info.num_subcores
    nw = nc * ns
    bpw = t // nw
    mesh = plsc.VectorSubcoreMesh(core_axis_name="c", subcore_axis_name="s")

    @functools.partial(
        pl.kernel, mesh=mesh,
        out_type=[jax.ShapeDtypeStruct((t, d), jnp.float32),
                  jax.ShapeDtypeStruct((t, d), jnp.float32)],
        scratch_types=[
            pltpu.VMEM((bpw, d), jnp.float32),
            pltpu.VMEM((bpw,), jnp.int32),
        ],
    )
    def k(yb_hbm, slots_hbm, g0_hbm, g1_hbm, rows_v, idx_v):
        wid = lax.axis_index("s") * nc + lax.axis_index("c")
        base = wid * bpw
        pltpu.sync_copy(slots_hbm.at[2, pl.ds(base, bpw)], idx_v)
        pltpu.sync_copy(yb_hbm.at[idx_v], rows_v)
        pltpu.sync_copy(rows_v, g0_hbm.at[pl.ds(base, bpw)])
        pltpu.sync_copy(slots_hbm.at[3, pl.ds(base, bpw)], idx_v)
        pltpu.sync_copy(yb_hbm.at[idx_v], rows_v)
        pltpu.sync_copy(rows_v, g1_hbm.at[pl.ds(base, bpw)])

    return k(yb, slots)


def _ffn_kernel(capb, loads_ref, xb_ref, w1_ref, w2_ref, b1_ref, b2_ref,
                yb_ref, w1c_ref, w2c_ref):
    e = pl.program_id(0)
    cb = pl.program_id(1)
    needed = loads_ref[e, 0] > cb * capb

    @pl.when(needed)
    def _():
        @pl.when(cb == 0)
        def _():
            w1c_ref[...] = w1_ref[0].astype(jnp.bfloat16)
            w2c_ref[...] = w2_ref[0].astype(jnp.bfloat16)

        xb = xb_ref[...].astype(jnp.bfloat16)
        hpre = jnp.dot(xb, w1c_ref[...],
                       preferred_element_type=jnp.float32) + b1_ref[0]
        h = (hpre * 0.5 * (1.0 + jax.lax.erf(hpre * 0.7071067811865476))
             ).astype(jnp.bfloat16)
        yb_ref[...] = jnp.dot(h, w2c_ref[...],
                              preferred_element_type=jnp.float32) + b2_ref[0]


def _expert_ffn(xb, w1, b1, w2, b2, loads, capp, capb):
    ne, d, dff = w1.shape
    nrows = ne * capp
    ncb = capp // capb
    b1r = b1.reshape(ne, 1, dff)
    b2r = b2.reshape(ne, 1, d)
    grid_spec = pltpu.PrefetchScalarGridSpec(
        num_scalar_prefetch=1,
        grid=(ne, ncb),
        in_specs=[
            pl.BlockSpec((capb, d),
                         lambda e, cb, L: (e * (capp // capb) + cb, 0)),
            pl.BlockSpec((1, d, dff),
                         lambda e, cb, L: (jnp.minimum(
                             e + (cb >= 1).astype(jnp.int32), ne - 1), 0, 0)),
            pl.BlockSpec((1, dff, d),
                         lambda e, cb, L: (jnp.minimum(
                             e + (cb >= 2).astype(jnp.int32), ne - 1), 0, 0)),
            pl.BlockSpec((1, 1, dff), lambda e, cb, L: (e, 0, 0)),
            pl.BlockSpec((1, 1, d), lambda e, cb, L: (e, 0, 0)),
        ],
        out_specs=pl.BlockSpec((capb, d),
                               lambda e, cb, L: (e * (capp // capb) + cb, 0)),
        scratch_shapes=[pltpu.VMEM((d, dff), jnp.bfloat16),
                        pltpu.VMEM((dff, d), jnp.bfloat16)],
    )
    return pl.pallas_call(
        functools.partial(_ffn_kernel, capb),
        grid_spec=grid_spec,
        out_shape=jax.ShapeDtypeStruct((nrows, d), jnp.float32),
    )(loads, xb, w1, w2, b1r, b2r)


def _fbc_kernel(nfb_ref, x_ref, wf1_hbm, wf2_hbm, bf1_ref, bf2_ref,
                g0_ref, g1_ref, wfb_ref, out_ref,
                wf1_v, wf2_v, flag_ref, sem1, sem2):
    tb = pl.program_id(0)
    w0 = wfb_ref[0, :][:, None]
    w1 = wfb_ref[1, :][:, None]
    fb = wfb_ref[2, :][:, None]
    g = g0_ref[...] * w0 + g1_ref[...] * w1

    @pl.when(tb == 0)
    def _():
        flag_ref[0] = 0

    @pl.when(nfb_ref[tb] > 0)
    def _():
        @pl.when(flag_ref[0] == 0)
        def _():
            cp1 = pltpu.make_async_copy(wf1_hbm, wf1_v, sem1)
            cp2 = pltpu.make_async_copy(wf2_hbm, wf2_v, sem2)
            cp1.start()
            cp2.start()
            cp1.wait()
            cp2.wait()
            flag_ref[0] = 1

        xb = x_ref[...].astype(jnp.bfloat16)
        hpre = jnp.dot(xb, wf1_v[...].astype(jnp.bfloat16),
                       preferred_element_type=jnp.float32) + bf1_ref[0]
        h = (hpre * 0.5 * (1.0 + jax.lax.erf(hpre * 0.7071067811865476))
             ).astype(jnp.bfloat16)
        yf = jnp.dot(h, wf2_v[...].astype(jnp.bfloat16),
                     preferred_element_type=jnp.float32) + bf2_ref[0]
        out_ref[...] = jnp.where(fb > 0.0, yf, g)

    @pl.when(nfb_ref[tb] == 0)
    def _():
        out_ref[...] = g


def _fallback_combine(x, wf1, bf1, wf2, bf2, g0, g1, wfb, nfb_blk, tblk):
    t, d = x.shape
    dff = wf1.shape[1]
    ntb = t // tblk
    bf1r = bf1.reshape(1, 1, dff)
    bf2r = bf2.reshape(1, 1, d)
    grid_spec = pltpu.PrefetchScalarGridSpec(
        num_scalar_prefetch=1,
        grid=(ntb,),
        in_specs=[
            pl.BlockSpec((tblk, d),
                         lambda tb, N: (jnp.where(N[tb] > 0, tb, 0), 0)),
            pl.BlockSpec(memory_space=pltpu.MemorySpace.HBM),
            pl.BlockSpec(memory_space=pltpu.MemorySpace.HBM),
            pl.BlockSpec((1, 1, dff), lambda tb, N: (0, 0, 0)),
            pl.BlockSpec((1, 1, d), lambda tb, N: (0, 0, 0)),
            pl.BlockSpec((tblk, d), lambda tb, N: (tb, 0)),
            pl.BlockSpec((tblk, d), lambda tb, N: (tb, 0)),
            pl.BlockSpec((3, tblk), lambda tb, N: (0, tb)),
        ],
        out_specs=pl.BlockSpec((tblk, d), lambda tb, N: (tb, 0)),
        scratch_shapes=[pltpu.VMEM((d, dff), jnp.float32),
                        pltpu.VMEM((dff, d), jnp.float32),
                        pltpu.SMEM((1,), jnp.int32),
                        pltpu.SemaphoreType.DMA,
                        pltpu.SemaphoreType.DMA],
    )
    return pl.pallas_call(
        _fbc_kernel,
        grid_spec=grid_spec,
        out_shape=jax.ShapeDtypeStruct((t, d), jnp.float32),
    )(nfb_blk, x, wf1, wf2, bf1r, bf2r, g0, g1, wfb)


def _moe_forward(x, w1, b1, w2, b2, wf1, bf1, wf2, bf2, routes):
    t, d = x.shape
    ne, _, dff = w1.shape
    k = routes.shape[1]
    cap = int(1.25 * math.ceil(t * k / ne))
    capp = 768
    capb = 256
    nwork = 32
    nrows = ne * capp
    nrows_pad = nrows + 256

    routesT = routes.astype(jnp.int32).T
    slots, wfb, loads = pl.pallas_call(
        functools.partial(_routing_kernel, cap, capp, nwork),
        out_shape=[jax.ShapeDtypeStruct((4, t), jnp.int32),
                   jax.ShapeDtypeStruct((3, t), jnp.float32),
                   jax.ShapeDtypeStruct((ne, 1), jnp.int32)],
    )(routesT)

    loads_i = loads
    tblk = 512
    nfb_blk = jnp.sum(wfb[2].reshape(t // tblk, tblk),
                      axis=1).astype(jnp.int32)

    xb = _sc_dispatch(nrows_pad, x, slots)
    yb = _expert_ffn(xb, w1, b1, w2, b2, loads_i, capp, capb)
    g0, g1 = _sc_gather2(yb, slots)
    out = _fallback_combine(x, wf1, bf1, wf2, bf2, g0, g1, wfb, nfb_blk,
                            tblk=tblk)
    return out


def kernel(x, W1, b1, W2, b2, Wf1, bf1, Wf2, bf2, routes):
    return _moe_forward(x, W1, b1, W2, b2, Wf1, bf1, Wf2, bf2, routes)
